# Initial kernel scaffold; baseline (speedup 1.0000x reference)
import jax
import jax.numpy as jnp
from jax.experimental import pallas as pl


def kernel(x, weight):
    raise NotImplementedError("write your pallas kernel here")


# speedup vs baseline: 46.8749x; 46.8749x over previous
import functools

import jax
import jax.numpy as jnp
import numpy as np
from jax import lax
from jax.experimental import pallas as pl
from jax.experimental.pallas import tpu as pltpu

N_WIDTH = 32
N_ORDER = 5
N_ELEMENTS = 10
N_NODES = N_ELEMENTS * N_ORDER + 1
N_SAMPLES = 2048
NDIM = 4
P = N_ORDER + 1
ROW = N_NODES * NDIM
FROW = N_WIDTH * ROW
DELTA_X = 0.5 * N_ORDER * 1.0 / (N_NODES - 1)


def _poly_coeffs():
    nodes = np.linspace(-1.0, 1.0, P)
    Ls, dLs, ddLs = [], [], []
    for c in range(P):
        p = np.poly1d([1.0])
        for m in range(P):
            if m != c:
                p = p * np.poly1d([1.0, -nodes[m]]) / (nodes[c] - nodes[m])
        Ls.append(p.coeffs)
        dLs.append(p.deriv(1).coeffs / DELTA_X)
        ddLs.append(p.deriv(2).coeffs / (DELTA_X ** 2))
    return (np.array(Ls, np.float32), np.array(dLs, np.float32),
            np.array(ddLs, np.float32))


_CL, _CD, _CDD = _poly_coeffs()


def _horner(coeffs_row, x):
    acc = jnp.full_like(x, coeffs_row[0])
    for c in coeffs_row[1:]:
        acc = acc * x + c
    return acc


def _body(bs, x_ref, wt_ref, phi_ref, dphi_ref, ddphi_ref,
          t_ref, dt_ref, ddt_ref):
    x = x_ref[...]
    lane = lax.broadcasted_iota(jnp.int32, (bs, ROW), 1)
    n_e = lane // NDIM
    j_e = lane - n_e * NDIM

    xc = [x[:, j:j + 1] for j in range(NDIM)]
    x_e = jnp.where(j_e == 0, xc[0],
          jnp.where(j_e == 1, xc[1],
          jnp.where(j_e == 2, xc[2], xc[3])))

    x_shift = (N_NODES - 1) * x_e
    id_elem = jnp.clip(jnp.floor(x_shift / N_ORDER), 0, N_ELEMENTS - 1)
    n0f = id_elem * N_ORDER
    x_t = 2.0 * (x_shift - n0f) / N_ORDER - 1.0
    q = n_e - n0f.astype(jnp.int32)

    phi = jnp.zeros((bs, ROW), jnp.float32)
    dphi = jnp.zeros((bs, ROW), jnp.float32)
    ddphi = jnp.zeros((bs, ROW), jnp.float32)
    for c in range(P):
        m = q == c
        phi = jnp.where(m, _horner(_CL[c], x_t), phi)
        dphi = jnp.where(m, _horner(_CD[c], x_t), dphi)
        ddphi = jnp.where(m, _horner(_CDD[c], x_t), ddphi)

    wt = wt_ref[...]
    t_ref[...] = jnp.dot(phi, wt, preferred_element_type=jnp.float32)
    dt_ref[...] = jnp.dot(dphi, wt, preferred_element_type=jnp.float32)
    ddt_ref[...] = jnp.dot(ddphi, wt, preferred_element_type=jnp.float32)

    for k in range(N_WIDTH):
        sl = pl.ds(k * ROW, ROW)
        phi_ref[:, sl] = phi
        dphi_ref[:, sl] = dphi
        ddphi_ref[:, sl] = ddphi


@jax.jit
def kernel(x, weight):
    S = x.shape[0]
    bs = 128
    grid = (S // bs,)
    wt = weight.reshape(N_WIDTH, ROW).T

    out_shapes = (
        jax.ShapeDtypeStruct((S, FROW), jnp.float32),
        jax.ShapeDtypeStruct((S, FROW), jnp.float32),
        jax.ShapeDtypeStruct((S, FROW), jnp.float32),
        jax.ShapeDtypeStruct((S, N_WIDTH), jnp.float32),
        jax.ShapeDtypeStruct((S, N_WIDTH), jnp.float32),
        jax.ShapeDtypeStruct((S, N_WIDTH), jnp.float32),
    )
    big = pl.BlockSpec((bs, FROW), lambda i: (i, 0))
    small = pl.BlockSpec((bs, N_WIDTH), lambda i: (i, 0))
    phi_f, dphi_f, ddphi_f, t, dt, ddt = pl.pallas_call(
        functools.partial(_body, bs),
        grid=grid,
        in_specs=[
            pl.BlockSpec((bs, NDIM), lambda i: (i, 0)),
            pl.BlockSpec((ROW, N_WIDTH), lambda i: (0, 0)),
        ],
        out_specs=(big, big, big, small, small, small),
        out_shape=out_shapes,
    )(x, wt)

    shp = (S, N_WIDTH, N_NODES, NDIM)
    return (t, dt, ddt,
            phi_f.reshape(shp), dphi_f.reshape(shp), ddphi_f.reshape(shp))


# speedup vs baseline: 351.4137x; 7.4968x over previous
import functools

import jax
import jax.numpy as jnp
import numpy as np
from jax import lax
from jax.experimental import pallas as pl
from jax.experimental.pallas import tpu as pltpu

N_WIDTH = 32
N_ORDER = 5
N_ELEMENTS = 10
N_NODES = N_ELEMENTS * N_ORDER + 1
NDIM = 4
P = N_ORDER + 1
ROW = N_NODES * NDIM
KN = N_WIDTH * N_NODES
DELTA_X = 0.5 * N_ORDER * 1.0 / (N_NODES - 1)


def _poly_coeffs():
    nodes = np.linspace(-1.0, 1.0, P)
    Ls, dLs, ddLs = [], [], []
    for c in range(P):
        p = np.poly1d([1.0])
        for m in range(P):
            if m != c:
                p = p * np.poly1d([1.0, -nodes[m]]) / (nodes[c] - nodes[m])
        Ls.append(p.coeffs)
        dLs.append(p.deriv(1).coeffs / DELTA_X)
        ddLs.append(p.deriv(2).coeffs / (DELTA_X ** 2))
    return (np.array(Ls, np.float32), np.array(dLs, np.float32),
            np.array(ddLs, np.float32))


_CL, _CD, _CDD = _poly_coeffs()


def _horner(coeffs_row, x):
    acc = jnp.full_like(x, coeffs_row[0])
    for c in coeffs_row[1:]:
        acc = acc * x + c
    return acc


def _body(bs, x_ref, w_ref, phi_ref, dphi_ref, ddphi_ref,
          t_ref, dt_ref, ddt_ref):
    x = x_ref[...]
    x_shift = (N_NODES - 1) * x
    id_elem = jnp.clip(jnp.floor(x_shift / N_ORDER), 0, N_ELEMENTS - 1)
    n0f = id_elem * N_ORDER
    x_t4 = 2.0 * (x_shift - n0f) / N_ORDER - 1.0

    r = lax.broadcasted_iota(jnp.int32, (ROW, bs), 0)
    n_e = r // NDIM
    j_e = r - n_e * NDIM

    def expand(a):
        return jnp.where(j_e == 0, a[0:1, :],
               jnp.where(j_e == 1, a[1:2, :],
               jnp.where(j_e == 2, a[2:3, :], a[3:4, :])))

    x_t = expand(x_t4)
    q = n_e - expand(n0f).astype(jnp.int32)

    phi = jnp.zeros((ROW, bs), jnp.float32)
    dphi = jnp.zeros((ROW, bs), jnp.float32)
    ddphi = jnp.zeros((ROW, bs), jnp.float32)
    for c in range(P):
        m = q == c
        phi = jnp.where(m, _horner(_CL[c], x_t), phi)
        dphi = jnp.where(m, _horner(_CD[c], x_t), dphi)
        ddphi = jnp.where(m, _horner(_CDD[c], x_t), ddphi)

    w = w_ref[...]
    t_ref[...] = jnp.dot(w, phi, preferred_element_type=jnp.float32)
    dt_ref[...] = jnp.dot(w, dphi, preferred_element_type=jnp.float32)
    ddt_ref[...] = jnp.dot(w, ddphi, preferred_element_type=jnp.float32)

    phi3 = phi.reshape(N_NODES, NDIM, bs)
    dphi3 = dphi.reshape(N_NODES, NDIM, bs)
    ddphi3 = ddphi.reshape(N_NODES, NDIM, bs)
    for k in range(N_WIDTH):
        sl = pl.ds(k * N_NODES, N_NODES)
        phi_ref[sl] = phi3
        dphi_ref[sl] = dphi3
        ddphi_ref[sl] = ddphi3


@jax.jit
def kernel(x, weight):
    S = x.shape[0]
    bs = 128
    grid = (S // bs,)
    xT = x.T
    wf = weight.reshape(N_WIDTH, ROW)

    out_shapes = (
        jax.ShapeDtypeStruct((KN, NDIM, S), jnp.float32),
        jax.ShapeDtypeStruct((KN, NDIM, S), jnp.float32),
        jax.ShapeDtypeStruct((KN, NDIM, S), jnp.float32),
        jax.ShapeDtypeStruct((N_WIDTH, S), jnp.float32),
        jax.ShapeDtypeStruct((N_WIDTH, S), jnp.float32),
        jax.ShapeDtypeStruct((N_WIDTH, S), jnp.float32),
    )
    big = pl.BlockSpec((KN, NDIM, bs), lambda i: (0, 0, i))
    small = pl.BlockSpec((N_WIDTH, bs), lambda i: (0, i))
    phi_t, dphi_t, ddphi_t, t_t, dt_t, ddt_t = pl.pallas_call(
        functools.partial(_body, bs),
        grid=grid,
        in_specs=[
            pl.BlockSpec((NDIM, bs), lambda i: (0, i)),
            pl.BlockSpec((N_WIDTH, ROW), lambda i: (0, 0)),
        ],
        out_specs=(big, big, big, small, small, small),
        out_shape=out_shapes,
    )(xT, wf)

    def untr(a):
        return a.reshape(N_WIDTH, N_NODES, NDIM, S).transpose(3, 0, 1, 2)

    return (t_t.T, dt_t.T, ddt_t.T, untr(phi_t), untr(dphi_t), untr(ddphi_t))
